# baseline (speedup 1.0000x reference)
import functools

import jax
import jax.numpy as jnp
from jax import lax
from jax.experimental import pallas as pl
from jax.experimental.pallas import tpu as pltpu
from jax.experimental.pallas import tpu_sc as plsc

DEPTH = 12
F = 128
N = 10000
E = 320000

NC = 2
NS = 16
NW = NC * NS
EPW = E // NW
B = 80
K = EPW // B
NPAD = 10240
NPT = NPAD // NS
FH = F // 2
EPT = E // NS
K2 = EPT // B

BE = 2000
GE = E // BE


def _sc_mesh():
    return plsc.VectorSubcoreMesh(
        core_axis_name="c", subcore_axis_name="s",
        num_cores=NC, num_subcores=NS)


def _sc_gather_sum(x3, x4, idx_s3, idx_d3):
    @functools.partial(
        pl.kernel,
        out_type=jax.ShapeDtypeStruct((E, F), jnp.float32),
        mesh=_sc_mesh(),
        scratch_types=[
            pltpu.VMEM((K, B), jnp.int32),
            pltpu.VMEM((K, B), jnp.int32),
            pltpu.VMEM((B, F), jnp.float32),
            pltpu.VMEM((B, F), jnp.float32),
        ],
    )
    def k(x3_hbm, x4_hbm, idxs_hbm, idxd_hbm, s_hbm, idxs_v, idxd_v, b3, b4):
        wid = lax.axis_index("s") * NC + lax.axis_index("c")
        base = wid * EPW
        pltpu.sync_copy(idxs_hbm.at[wid], idxs_v)
        pltpu.sync_copy(idxd_hbm.at[wid], idxd_v)

        def blk(j, _):
            pltpu.sync_copy(x3_hbm.at[idxs_v.at[j]], b3)
            pltpu.sync_copy(x4_hbm.at[idxd_v.at[j]], b4)

            def row(i, _):
                for l in range(F // 16):
                    sl = pl.ds(l * 16, 16)
                    b3[i, sl] = b3[i, sl] + b4[i, sl]
                return 0

            lax.fori_loop(0, B, row, 0)
            pltpu.sync_copy(b3, s_hbm.at[pl.ds(base + j * B, B)])
            return 0

        lax.fori_loop(0, K, blk, 0)

    return k(x3, x4, idx_s3, idx_d3)


def _sc_gate_scatter(x2h, w, idx_s2, idx_d2, zeros_n):
    @functools.partial(
        pl.kernel,
        out_type=jax.ShapeDtypeStruct((NC, NPAD, FH), jnp.float32),
        mesh=_sc_mesh(),
        compiler_params=pltpu.CompilerParams(use_tc_tiling_on_sc=False),
        scratch_types=[
            pltpu.VMEM((K2, B), jnp.int32),
            pltpu.VMEM((K2, B), jnp.int32),
            pltpu.VMEM((B, FH), jnp.float32),
            pltpu.VMEM((B, F), jnp.float32),
            pltpu.VMEM_SHARED((NPAD, FH), jnp.float32),
        ],
    )
    def k(x2_hbm, w_hbm, idxs_hbm, idxd_hbm, z_hbm, acc_hbm,
          idxs_v, idxd_v, bx, bw, acc_sh):
        cid = lax.axis_index("c")
        sid = lax.axis_index("s")
        base = sid * EPT
        rows = pl.ds(sid * NPT, NPT)
        pltpu.sync_copy(z_hbm.at[pl.ds(sid * NPT, NPT)], acc_sh.at[rows])
        pltpu.sync_copy(idxs_hbm.at[sid], idxs_v)
        pltpu.sync_copy(idxd_hbm.at[sid], idxd_v)
        plsc.subcore_barrier()
        cbase = cid * FH

        def blk(j, _):
            pltpu.sync_copy(x2_hbm.at[cid].at[idxd_v.at[j]], bx)
            pltpu.sync_copy(w_hbm.at[pl.ds(base + j * B, B)], bw)

            def row(i, _):
                for l in range(FH // 16):
                    sl = pl.ds(l * 16, 16)
                    wv = bw[i, pl.ds(cbase + l * 16, 16)]
                    bx[i, sl] = bx[i, sl] / (1.0 + jnp.exp(-wv))
                return 0

            lax.fori_loop(0, B, row, 0)
            pltpu.sync_copy(bx, acc_sh.at[idxs_v.at[j]], add=True)
            return 0

        lax.fori_loop(0, K2, blk, 0)
        plsc.subcore_barrier()
        pltpu.sync_copy(acc_sh.at[rows], acc_hbm.at[cid, rows])

    return k(x2h, w, idx_s2, idx_d2, zeros_n)


def _sc_counts(idx_s2, ones_b, zeros_h):
    @functools.partial(
        pl.kernel,
        out_type=jax.ShapeDtypeStruct((NC, NPAD, FH), jnp.float32),
        mesh=_sc_mesh(),
        compiler_params=pltpu.CompilerParams(use_tc_tiling_on_sc=False),
        scratch_types=[
            pltpu.VMEM((K2, B), jnp.int32),
            pltpu.VMEM((B, FH), jnp.float32),
            pltpu.VMEM_SHARED((NPAD, FH), jnp.float32),
        ],
    )
    def k(idxs_hbm, ones_hbm, z_hbm, acc_hbm, idxs_v, bones, acc_sh):
        cid = lax.axis_index("c")
        sid = lax.axis_index("s")
        rows = pl.ds(sid * NPT, NPT)
        pltpu.sync_copy(z_hbm.at[rows], acc_sh.at[rows])
        pltpu.sync_copy(idxs_hbm.at[sid], idxs_v)
        pltpu.sync_copy(ones_hbm, bones)
        plsc.subcore_barrier()

        def blk(j, _):
            pltpu.sync_copy(bones, acc_sh.at[idxs_v.at[j]], add=True)
            return 0

        lax.fori_loop(0, K2, blk, 0)
        plsc.subcore_barrier()
        pltpu.sync_copy(acc_sh.at[rows], acc_hbm.at[cid, rows])

    return k(idx_s2, ones_b, zeros_h)


def _tc_init_node(x, W, b):
    def body(x_ref, w_ref, b_ref, o_ref):
        h = jnp.dot(x_ref[...], w_ref[...],
                    preferred_element_type=jnp.float32) + b_ref[...]
        o_ref[...] = h * jax.nn.sigmoid(h)

    return pl.pallas_call(
        body,
        out_shape=jax.ShapeDtypeStruct((N, F), jnp.float32),
    )(x, W, b.reshape(1, F))


def _tc_init_edge(ea, W, b):
    def body(ea_ref, w_ref, b_ref, o_ref):
        h = jnp.dot(ea_ref[...], w_ref[...],
                    preferred_element_type=jnp.float32) + b_ref[...]
        o_ref[...] = h * jax.nn.sigmoid(h)

    ef = ea.shape[1]
    return pl.pallas_call(
        body,
        grid=(GE,),
        in_specs=[
            pl.BlockSpec((BE, ef), lambda i: (i, 0)),
            pl.BlockSpec((ef, F), lambda i: (0, 0)),
            pl.BlockSpec((1, F), lambda i: (0, 0)),
        ],
        out_specs=pl.BlockSpec((BE, F), lambda i: (i, 0)),
        out_shape=jax.ShapeDtypeStruct((E, F), jnp.float32),
    )(ea, W, b.reshape(1, F))


def _tc_node_mm(x, Wcat, bcat):
    def body(x_ref, w_ref, b_ref, o1, o2, o4, o3):
        h = jnp.dot(x_ref[...], w_ref[...],
                    preferred_element_type=jnp.float32) + b_ref[...]
        o1[...] = h[:, 0 * F:1 * F]
        o2[0] = h[:, F:F + FH]
        o2[1] = h[:, F + FH:2 * F]
        o4[...] = h[:, 2 * F:3 * F]
        o3[...] = h[:, 3 * F:4 * F]

    sh = jax.ShapeDtypeStruct((N, F), jnp.float32)
    sh2 = jax.ShapeDtypeStruct((NC, N, FH), jnp.float32)
    return pl.pallas_call(
        body,
        out_shape=(sh, sh2, sh, sh),
    )(x, Wcat, bcat.reshape(1, 4 * F))


def _tc_edge_pass1(w, s, E0Wi, E0bi):
    def body(w_ref, s_ref, ew_ref, eb_ref, h_ref, st_ref):
        h = jnp.dot(w_ref[...], ew_ref[...],
                    preferred_element_type=jnp.float32)
        h = h + eb_ref[...] + s_ref[...]
        h_ref[...] = h

        @pl.when(pl.program_id(0) == 0)
        def _():
            st_ref[...] = jnp.zeros_like(st_ref)

        st_ref[0:1, :] += jnp.sum(h, axis=0, keepdims=True)
        st_ref[1:2, :] += jnp.sum(h * h, axis=0, keepdims=True)

    return pl.pallas_call(
        body,
        grid=(GE,),
        in_specs=[
            pl.BlockSpec((BE, F), lambda i: (i, 0)),
            pl.BlockSpec((BE, F), lambda i: (i, 0)),
            pl.BlockSpec((F, F), lambda i: (0, 0)),
            pl.BlockSpec((1, F), lambda i: (0, 0)),
        ],
        out_specs=(
            pl.BlockSpec((BE, F), lambda i: (i, 0)),
            pl.BlockSpec((8, F), lambda i: (0, 0)),
        ),
        out_shape=(
            jax.ShapeDtypeStruct((E, F), jnp.float32),
            jax.ShapeDtypeStruct((8, F), jnp.float32),
        ),
    )(w, s, E0Wi, E0bi.reshape(1, F))


def _tc_inv_deg(cnt):
    def body(c_ref, o_ref):
        o_ref[...] = 1.0 / jnp.maximum(c_ref[0, :N], 1.0)

    return pl.pallas_call(
        body,
        out_shape=jax.ShapeDtypeStruct((N, FH), jnp.float32),
    )(cnt)


def _tc_node_update(x0, x1, acc, inv, g, b):
    def body(x0_ref, x1_ref, a_ref, i_ref, g_ref, b_ref, o_ref):
        agg = jnp.concatenate(
            [a_ref[0, :N] * i_ref[...], a_ref[1, :N] * i_ref[...]], axis=1)
        t = x1_ref[...] + agg
        m = jnp.mean(t, axis=0, keepdims=True)
        v = jnp.mean(t * t, axis=0, keepdims=True) - m * m
        y = (t - m) * lax.rsqrt(v + 1e-5) * g_ref[...] + b_ref[...]
        o_ref[...] = x0_ref[...] + y * jax.nn.sigmoid(y)

    return pl.pallas_call(
        body,
        out_shape=jax.ShapeDtypeStruct((N, F), jnp.float32),
    )(x0, x1, acc, inv, g.reshape(1, F), b.reshape(1, F))


def _tc_edge_update(w0, h, st, g, b):
    def body(w_ref, h_ref, st_ref, g_ref, b_ref, o_ref):
        m = st_ref[0:1, :] * (1.0 / E)
        v = st_ref[1:2, :] * (1.0 / E) - m * m
        y = (h_ref[...] - m) * lax.rsqrt(v + 1e-5) * g_ref[...] + b_ref[...]
        o_ref[...] = w_ref[...] + y * jax.nn.sigmoid(y)

    return pl.pallas_call(
        body,
        grid=(GE,),
        in_specs=[
            pl.BlockSpec((BE, F), lambda i: (i, 0)),
            pl.BlockSpec((BE, F), lambda i: (i, 0)),
            pl.BlockSpec((8, F), lambda i: (0, 0)),
            pl.BlockSpec((1, F), lambda i: (0, 0)),
            pl.BlockSpec((1, F), lambda i: (0, 0)),
        ],
        out_specs=pl.BlockSpec((BE, F), lambda i: (i, 0)),
        out_shape=jax.ShapeDtypeStruct((E, F), jnp.float32),
    )(w0, h, st, g.reshape(1, F), b.reshape(1, F))


def kernel(x, edge_index, edge_attr, v_lin0_W, v_lin0_b, V1_W, V1_b, V2_W,
           V2_b, V3_W, V3_b, V4_W, V4_b, vbn_g, vbn_b, e_lin0_W, e_lin0_b,
           E0_W, E0_b, ebn_g, ebn_b):
    src = edge_index[0]
    dst = edge_index[1]
    idx_s3 = src.reshape(NW, K, B)
    idx_d3 = dst.reshape(NW, K, B)
    idx_s2 = src.reshape(NS, K2, B)
    idx_d2 = dst.reshape(NS, K2, B)
    zeros_h = jnp.zeros((NPAD, FH), jnp.float32)
    ones_b = jnp.ones((B, FH), jnp.float32)

    Wcat = jnp.concatenate([V1_W, V2_W, V4_W, V3_W], axis=2)
    bcat = jnp.concatenate([V1_b, V2_b, V4_b, V3_b], axis=1)

    cnt = _sc_counts(idx_s2, ones_b, zeros_h)
    inv = _tc_inv_deg(cnt)

    xc = _tc_init_node(x, v_lin0_W, v_lin0_b)
    wc = _tc_init_edge(edge_attr, e_lin0_W, e_lin0_b)

    for i in range(DEPTH):
        x1, x2h, x4, x3 = _tc_node_mm(xc, Wcat[i], bcat[i])
        s = _sc_gather_sum(x3, x4, idx_s3, idx_d3)
        acc = _sc_gate_scatter(x2h, wc, idx_s2, idx_d2, zeros_h)
        h, st = _tc_edge_pass1(wc, s, E0_W[i], E0_b[i])
        xc = _tc_node_update(xc, x1, acc, inv, vbn_g[i], vbn_b[i])
        wc = _tc_edge_update(wc, h, st, ebn_g[i], ebn_b[i])
    return (xc, wc)


# speedup vs baseline: 2.5486x; 2.5486x over previous
import functools

import jax
import jax.numpy as jnp
from jax import lax
from jax.experimental import pallas as pl
from jax.experimental.pallas import tpu as pltpu
from jax.experimental.pallas import tpu_sc as plsc

DEPTH = 12
F = 128
N = 10000
E = 320000

NC = 2
NS = 16
NW = NC * NS
EPW = E // NW
B = 80
K = EPW // B
NPAD = 10240
NPT = NPAD // NS
FH = F // 2
EPT = E // NS
K2 = EPT // B

BE = 2000
GE = E // BE


def _sc_mesh():
    return plsc.VectorSubcoreMesh(
        core_axis_name="c", subcore_axis_name="s",
        num_cores=NC, num_subcores=NS)


def _sc_gather_sum(x3, x4, idx_s3, idx_d3):
    @functools.partial(
        pl.kernel,
        out_type=jax.ShapeDtypeStruct((E, F), jnp.float32),
        mesh=_sc_mesh(),
        scratch_types=[
            pltpu.VMEM((K, B), jnp.int32),
            pltpu.VMEM((K, B), jnp.int32),
            [pltpu.VMEM((B, F), jnp.float32)] * 2,
            [pltpu.VMEM((B, F), jnp.float32)] * 2,
            [pltpu.VMEM((B, F), jnp.float32)] * 2,
            [pltpu.SemaphoreType.DMA] * 2,
            [pltpu.SemaphoreType.DMA] * 2,
        ],
    )
    def k(x3_hbm, x4_hbm, idxs_hbm, idxd_hbm, s_hbm, idxs_v, idxd_v,
          b3, b4, bo, sg, sw):
        wid = lax.axis_index("s") * NC + lax.axis_index("c")
        base = wid * EPW
        pltpu.sync_copy(idxs_hbm.at[wid], idxs_v)
        pltpu.sync_copy(idxd_hbm.at[wid], idxd_v)

        def gather(j, t):
            pltpu.async_copy(x3_hbm.at[idxs_v.at[j]], b3[t], sg[t])
            pltpu.async_copy(x4_hbm.at[idxd_v.at[j]], b4[t], sg[t])

        def process(j, t):
            pltpu.make_async_copy(x3_hbm.at[idxs_v.at[0]], b3[t], sg[t]).wait()
            pltpu.make_async_copy(x4_hbm.at[idxd_v.at[0]], b4[t], sg[t]).wait()

            @pl.when(j >= 2)
            def _():
                pltpu.make_async_copy(bo[t], s_hbm.at[pl.ds(0, B)],
                                      sw[t]).wait()

            def row(i, _):
                for l in range(F // 16):
                    sl = pl.ds(l * 16, 16)
                    bo[t][i, sl] = b3[t][i, sl] + b4[t][i, sl]
                return 0

            lax.fori_loop(0, B, row, 0)
            pltpu.async_copy(bo[t], s_hbm.at[pl.ds(base + j * B, B)], sw[t])

            @pl.when(j + 2 < K)
            def _():
                gather(j + 2, t)

        gather(0, 0)
        gather(1, 1)

        def pair(p, _):
            process(2 * p, 0)
            process(2 * p + 1, 1)
            return 0

        lax.fori_loop(0, K // 2, pair, 0)
        process(jnp.int32(K - 1), 0)
        pltpu.make_async_copy(bo[0], s_hbm.at[pl.ds(0, B)], sw[0]).wait()
        pltpu.make_async_copy(bo[1], s_hbm.at[pl.ds(0, B)], sw[1]).wait()

    return k(x3, x4, idx_s3, idx_d3)


def _sc_gate_scatter(x2h, gate, idx_s2, idx_d2, zeros_n):
    @functools.partial(
        pl.kernel,
        out_type=jax.ShapeDtypeStruct((NC, NPAD, FH), jnp.float32),
        mesh=_sc_mesh(),
        compiler_params=pltpu.CompilerParams(use_tc_tiling_on_sc=False),
        scratch_types=[
            pltpu.VMEM((K2, B), jnp.int32),
            pltpu.VMEM((K2, B), jnp.int32),
            [pltpu.VMEM((B, FH), jnp.float32)] * 2,
            [pltpu.VMEM((B, FH), jnp.float32)] * 2,
            [pltpu.VMEM((B, FH), jnp.float32)] * 2,
            pltpu.VMEM_SHARED((NPAD, FH), jnp.float32),
            [pltpu.SemaphoreType.DMA] * 2,
            [pltpu.SemaphoreType.DMA] * 2,
        ],
    )
    def k(x2_hbm, g_hbm, idxs_hbm, idxd_hbm, z_hbm, acc_hbm,
          idxs_v, idxd_v, bx, bw, bs, acc_sh, sg, ss):
        cid = lax.axis_index("c")
        sid = lax.axis_index("s")
        base = sid * EPT
        rows = pl.ds(sid * NPT, NPT)
        pltpu.sync_copy(z_hbm.at[rows], acc_sh.at[rows])
        pltpu.sync_copy(idxs_hbm.at[sid], idxs_v)
        pltpu.sync_copy(idxd_hbm.at[sid], idxd_v)
        plsc.subcore_barrier()

        def gather(j, t):
            pltpu.async_copy(x2_hbm.at[cid].at[idxd_v.at[j]], bx[t], sg[t])
            pltpu.async_copy(g_hbm.at[cid, pl.ds(base + j * B, B)],
                             bw[t], sg[t])

        def process(j, t):
            pltpu.make_async_copy(x2_hbm.at[cid].at[idxd_v.at[0]],
                                  bx[t], sg[t]).wait()
            pltpu.make_async_copy(g_hbm.at[cid, pl.ds(0, B)],
                                  bw[t], sg[t]).wait()

            @pl.when(j >= 2)
            def _():
                pltpu.make_async_copy(bs[t], acc_sh.at[idxs_v.at[0]],
                                      ss[t]).wait()

            def row(i, _):
                for l in range(FH // 16):
                    sl = pl.ds(l * 16, 16)
                    bs[t][i, sl] = bx[t][i, sl] * bw[t][i, sl]
                return 0

            lax.fori_loop(0, B, row, 0)

            @pl.when(j + 2 < K2)
            def _():
                gather(j + 2, t)

            pltpu.async_copy(bs[t], acc_sh.at[idxs_v.at[j]], ss[t], add=True)

        gather(0, 0)
        gather(1, 1)

        def pair(p, _):
            process(2 * p, 0)
            process(2 * p + 1, 1)
            return 0

        lax.fori_loop(0, K2 // 2, pair, 0)
        pltpu.make_async_copy(bs[0], acc_sh.at[idxs_v.at[0]], ss[0]).wait()
        pltpu.make_async_copy(bs[1], acc_sh.at[idxs_v.at[0]], ss[1]).wait()
        plsc.subcore_barrier()
        pltpu.sync_copy(acc_sh.at[rows], acc_hbm.at[cid, rows])

    return k(x2h, gate, idx_s2, idx_d2, zeros_n)


def _sc_counts(idx_s2, ones_b, zeros_h):
    @functools.partial(
        pl.kernel,
        out_type=jax.ShapeDtypeStruct((NC, NPAD, FH), jnp.float32),
        mesh=_sc_mesh(),
        compiler_params=pltpu.CompilerParams(use_tc_tiling_on_sc=False),
        scratch_types=[
            pltpu.VMEM((K2, B), jnp.int32),
            pltpu.VMEM((B, FH), jnp.float32),
            pltpu.VMEM_SHARED((NPAD, FH), jnp.float32),
            pltpu.SemaphoreType.DMA,
        ],
    )
    def k(idxs_hbm, ones_hbm, z_hbm, acc_hbm, idxs_v, bones, acc_sh, ss):
        cid = lax.axis_index("c")
        sid = lax.axis_index("s")
        rows = pl.ds(sid * NPT, NPT)
        pltpu.sync_copy(z_hbm.at[rows], acc_sh.at[rows])
        pltpu.sync_copy(idxs_hbm.at[sid], idxs_v)
        pltpu.sync_copy(ones_hbm, bones)
        plsc.subcore_barrier()

        def blk(j, _):
            pltpu.async_copy(bones, acc_sh.at[idxs_v.at[j]], ss, add=True)
            return 0

        lax.fori_loop(0, K2, blk, 0)

        def drain(j, _):
            pltpu.make_async_copy(bones, acc_sh.at[idxs_v.at[0]], ss).wait()
            return 0

        lax.fori_loop(0, K2, drain, 0)
        plsc.subcore_barrier()
        pltpu.sync_copy(acc_sh.at[rows], acc_hbm.at[cid, rows])

    return k(idx_s2, ones_b, zeros_h)


def _tc_init_node(x, W, b):
    def body(x_ref, w_ref, b_ref, o_ref):
        h = jnp.dot(x_ref[...], w_ref[...],
                    preferred_element_type=jnp.float32) + b_ref[...]
        o_ref[...] = h * jax.nn.sigmoid(h)

    return pl.pallas_call(
        body,
        out_shape=jax.ShapeDtypeStruct((N, F), jnp.float32),
    )(x, W, b.reshape(1, F))


def _tc_init_edge(ea, W, b):
    def body(ea_ref, w_ref, b_ref, o_ref, g_ref):
        h = jnp.dot(ea_ref[...], w_ref[...],
                    preferred_element_type=jnp.float32) + b_ref[...]
        o = h * jax.nn.sigmoid(h)
        o_ref[...] = o
        sg = jax.nn.sigmoid(o)
        g_ref[0] = sg[:, :FH]
        g_ref[1] = sg[:, FH:]

    ef = ea.shape[1]
    return pl.pallas_call(
        body,
        grid=(GE,),
        in_specs=[
            pl.BlockSpec((BE, ef), lambda i: (i, 0)),
            pl.BlockSpec((ef, F), lambda i: (0, 0)),
            pl.BlockSpec((1, F), lambda i: (0, 0)),
        ],
        out_specs=(
            pl.BlockSpec((BE, F), lambda i: (i, 0)),
            pl.BlockSpec((NC, BE, FH), lambda i: (0, i, 0)),
        ),
        out_shape=(
            jax.ShapeDtypeStruct((E, F), jnp.float32),
            jax.ShapeDtypeStruct((NC, E, FH), jnp.float32),
        ),
    )(ea, W, b.reshape(1, F))


def _tc_node_mm(x, Wcat, bcat):
    def body(x_ref, w_ref, b_ref, o1, o2, o4, o3):
        h = jnp.dot(x_ref[...], w_ref[...],
                    preferred_element_type=jnp.float32) + b_ref[...]
        o1[...] = h[:, 0 * F:1 * F]
        o2[0] = h[:, F:F + FH]
        o2[1] = h[:, F + FH:2 * F]
        o4[...] = h[:, 2 * F:3 * F]
        o3[...] = h[:, 3 * F:4 * F]

    sh = jax.ShapeDtypeStruct((N, F), jnp.float32)
    sh2 = jax.ShapeDtypeStruct((NC, N, FH), jnp.float32)
    return pl.pallas_call(
        body,
        out_shape=(sh, sh2, sh, sh),
    )(x, Wcat, bcat.reshape(1, 4 * F))


def _tc_edge_pass1(w, s, E0Wi, E0bi):
    def body(w_ref, s_ref, ew_ref, eb_ref, h_ref, st_ref):
        h = jnp.dot(w_ref[...], ew_ref[...],
                    preferred_element_type=jnp.float32)
        h = h + eb_ref[...] + s_ref[...]
        h_ref[...] = h

        @pl.when(pl.program_id(0) == 0)
        def _():
            st_ref[...] = jnp.zeros_like(st_ref)

        st_ref[0:1, :] += jnp.sum(h, axis=0, keepdims=True)
        st_ref[1:2, :] += jnp.sum(h * h, axis=0, keepdims=True)

    return pl.pallas_call(
        body,
        grid=(GE,),
        in_specs=[
            pl.BlockSpec((BE, F), lambda i: (i, 0)),
            pl.BlockSpec((BE, F), lambda i: (i, 0)),
            pl.BlockSpec((F, F), lambda i: (0, 0)),
            pl.BlockSpec((1, F), lambda i: (0, 0)),
        ],
        out_specs=(
            pl.BlockSpec((BE, F), lambda i: (i, 0)),
            pl.BlockSpec((8, F), lambda i: (0, 0)),
        ),
        out_shape=(
            jax.ShapeDtypeStruct((E, F), jnp.float32),
            jax.ShapeDtypeStruct((8, F), jnp.float32),
        ),
    )(w, s, E0Wi, E0bi.reshape(1, F))


def _tc_inv_deg(cnt):
    def body(c_ref, o_ref):
        o_ref[...] = 1.0 / jnp.maximum(c_ref[0, :N], 1.0)

    return pl.pallas_call(
        body,
        out_shape=jax.ShapeDtypeStruct((N, FH), jnp.float32),
    )(cnt)


def _tc_node_update(x0, x1, acc, inv, g, b):
    def body(x0_ref, x1_ref, a_ref, i_ref, g_ref, b_ref, o_ref):
        agg = jnp.concatenate(
            [a_ref[0, :N] * i_ref[...], a_ref[1, :N] * i_ref[...]], axis=1)
        t = x1_ref[...] + agg
        m = jnp.mean(t, axis=0, keepdims=True)
        v = jnp.mean(t * t, axis=0, keepdims=True) - m * m
        y = (t - m) * lax.rsqrt(v + 1e-5) * g_ref[...] + b_ref[...]
        o_ref[...] = x0_ref[...] + y * jax.nn.sigmoid(y)

    return pl.pallas_call(
        body,
        out_shape=jax.ShapeDtypeStruct((N, F), jnp.float32),
    )(x0, x1, acc, inv, g.reshape(1, F), b.reshape(1, F))


def _tc_edge_update(w0, h, st, g, b, want_gate):
    def body(w_ref, h_ref, st_ref, g_ref, b_ref, o_ref, *rest):
        m = st_ref[0:1, :] * (1.0 / E)
        v = st_ref[1:2, :] * (1.0 / E) - m * m
        y = (h_ref[...] - m) * lax.rsqrt(v + 1e-5) * g_ref[...] + b_ref[...]
        o = w_ref[...] + y * jax.nn.sigmoid(y)
        o_ref[...] = o
        if rest:
            sg = jax.nn.sigmoid(o)
            rest[0][0] = sg[:, :FH]
            rest[0][1] = sg[:, FH:]

    out_specs = [pl.BlockSpec((BE, F), lambda i: (i, 0))]
    out_shape = [jax.ShapeDtypeStruct((E, F), jnp.float32)]
    if want_gate:
        out_specs.append(pl.BlockSpec((NC, BE, FH), lambda i: (0, i, 0)))
        out_shape.append(jax.ShapeDtypeStruct((NC, E, FH), jnp.float32))
    res = pl.pallas_call(
        body,
        grid=(GE,),
        in_specs=[
            pl.BlockSpec((BE, F), lambda i: (i, 0)),
            pl.BlockSpec((BE, F), lambda i: (i, 0)),
            pl.BlockSpec((8, F), lambda i: (0, 0)),
            pl.BlockSpec((1, F), lambda i: (0, 0)),
            pl.BlockSpec((1, F), lambda i: (0, 0)),
        ],
        out_specs=tuple(out_specs),
        out_shape=tuple(out_shape),
    )(w0, h, st, g.reshape(1, F), b.reshape(1, F))
    return res if want_gate else (res[0], None)


def kernel(x, edge_index, edge_attr, v_lin0_W, v_lin0_b, V1_W, V1_b, V2_W,
           V2_b, V3_W, V3_b, V4_W, V4_b, vbn_g, vbn_b, e_lin0_W, e_lin0_b,
           E0_W, E0_b, ebn_g, ebn_b):
    src = edge_index[0]
    dst = edge_index[1]
    idx_s3 = src.reshape(NW, K, B)
    idx_d3 = dst.reshape(NW, K, B)
    idx_s2 = src.reshape(NS, K2, B)
    idx_d2 = dst.reshape(NS, K2, B)
    zeros_h = jnp.zeros((NPAD, FH), jnp.float32)
    ones_b = jnp.ones((B, FH), jnp.float32)

    Wcat = jnp.concatenate([V1_W, V2_W, V4_W, V3_W], axis=2)
    bcat = jnp.concatenate([V1_b, V2_b, V4_b, V3_b], axis=1)

    cnt = _sc_counts(idx_s2, ones_b, zeros_h)
    inv = _tc_inv_deg(cnt)

    xc = _tc_init_node(x, v_lin0_W, v_lin0_b)
    wc, gc = _tc_init_edge(edge_attr, e_lin0_W, e_lin0_b)

    for i in range(DEPTH):
        x1, x2h, x4, x3 = _tc_node_mm(xc, Wcat[i], bcat[i])
        s = _sc_gather_sum(x3, x4, idx_s3, idx_d3)
        acc = _sc_gate_scatter(x2h, gc, idx_s2, idx_d2, zeros_h)
        h, st = _tc_edge_pass1(wc, s, E0_W[i], E0_b[i])
        xc = _tc_node_update(xc, x1, acc, inv, vbn_g[i], vbn_b[i])
        wc, gc = _tc_edge_update(wc, h, st, ebn_g[i], ebn_b[i],
                                 want_gate=(i + 1 < DEPTH))
    return (xc, wc)


# speedup vs baseline: 2.5889x; 1.0158x over previous
import functools

import jax
import jax.numpy as jnp
from jax import lax
from jax.experimental import pallas as pl
from jax.experimental.pallas import tpu as pltpu
from jax.experimental.pallas import tpu_sc as plsc

DEPTH = 12
F = 128
N = 10000
E = 320000

NC = 2
NS = 16
NW = NC * NS
EPW = E // NW
B = 80
K = EPW // B
NPAD = 10240
NPT = NPAD // NS
FH = F // 2
EPT = E // NS
K2 = EPT // B

BE = 2000
GE = E // BE


def _sc_mesh():
    return plsc.VectorSubcoreMesh(
        core_axis_name="c", subcore_axis_name="s",
        num_cores=NC, num_subcores=NS)


def _sc_gather_sum(x3, x4, idx_s3, idx_d3):
    @functools.partial(
        pl.kernel,
        out_type=jax.ShapeDtypeStruct((E, F), jnp.float32),
        mesh=_sc_mesh(),
        scratch_types=[
            pltpu.VMEM((K, B), jnp.int32),
            pltpu.VMEM((K, B), jnp.int32),
            [pltpu.VMEM((B, F), jnp.float32)] * 2,
            [pltpu.VMEM((B, F), jnp.float32)] * 2,
            [pltpu.VMEM((B, F), jnp.float32)] * 2,
            [pltpu.SemaphoreType.DMA] * 2,
            [pltpu.SemaphoreType.DMA] * 2,
        ],
    )
    def k(x3_hbm, x4_hbm, idxs_hbm, idxd_hbm, s_hbm, idxs_v, idxd_v,
          b3, b4, bo, sg, sw):
        wid = lax.axis_index("s") * NC + lax.axis_index("c")
        base = wid * EPW
        pltpu.sync_copy(idxs_hbm.at[wid], idxs_v)
        pltpu.sync_copy(idxd_hbm.at[wid], idxd_v)

        def gather(j, t):
            pltpu.async_copy(x3_hbm.at[idxs_v.at[j]], b3[t], sg[t])
            pltpu.async_copy(x4_hbm.at[idxd_v.at[j]], b4[t], sg[t])

        def process(j, t):
            pltpu.make_async_copy(x3_hbm.at[idxs_v.at[0]], b3[t], sg[t]).wait()
            pltpu.make_async_copy(x4_hbm.at[idxd_v.at[0]], b4[t], sg[t]).wait()

            @pl.when(j >= 2)
            def _():
                pltpu.make_async_copy(bo[t], s_hbm.at[pl.ds(0, B)],
                                      sw[t]).wait()

            def row(i, _):
                for l in range(F // 16):
                    sl = pl.ds(l * 16, 16)
                    bo[t][i, sl] = b3[t][i, sl] + b4[t][i, sl]
                return 0

            lax.fori_loop(0, B, row, 0)
            pltpu.async_copy(bo[t], s_hbm.at[pl.ds(base + j * B, B)], sw[t])

            @pl.when(j + 2 < K)
            def _():
                gather(j + 2, t)

        gather(0, 0)
        gather(1, 1)

        def pair(p, _):
            process(2 * p, 0)
            process(2 * p + 1, 1)
            return 0

        lax.fori_loop(0, K // 2, pair, 0)
        process(jnp.int32(K - 1), 0)
        pltpu.make_async_copy(bo[0], s_hbm.at[pl.ds(0, B)], sw[0]).wait()
        pltpu.make_async_copy(bo[1], s_hbm.at[pl.ds(0, B)], sw[1]).wait()

    return k(x3, x4, idx_s3, idx_d3)


def _sc_gate_scatter(x2h, gate, idx_s2, idx_d2, zeros_n):
    @functools.partial(
        pl.kernel,
        out_type=jax.ShapeDtypeStruct((NC, NPAD, FH), jnp.float32),
        mesh=_sc_mesh(),
        compiler_params=pltpu.CompilerParams(use_tc_tiling_on_sc=False),
        scratch_types=[
            pltpu.VMEM((K2, B), jnp.int32),
            pltpu.VMEM((K2, B), jnp.int32),
            [pltpu.VMEM((B, FH), jnp.float32)] * 2,
            [pltpu.VMEM((B, FH), jnp.float32)] * 2,
            [pltpu.VMEM((B, FH), jnp.float32)] * 2,
            pltpu.VMEM_SHARED((NPAD, FH), jnp.float32),
            [pltpu.SemaphoreType.DMA] * 2,
            [pltpu.SemaphoreType.DMA] * 2,
        ],
    )
    def k(x2_hbm, g_hbm, idxs_hbm, idxd_hbm, z_hbm, acc_hbm,
          idxs_v, idxd_v, bx, bw, bs, acc_sh, sg, ss):
        cid = lax.axis_index("c")
        sid = lax.axis_index("s")
        base = sid * EPT
        rows = pl.ds(sid * NPT, NPT)
        pltpu.sync_copy(z_hbm.at[rows], acc_sh.at[rows])
        pltpu.sync_copy(idxs_hbm.at[sid], idxs_v)
        pltpu.sync_copy(idxd_hbm.at[sid], idxd_v)
        plsc.subcore_barrier()

        def gather(j, t):
            pltpu.async_copy(x2_hbm.at[cid].at[idxd_v.at[j]], bx[t], sg[t])
            pltpu.async_copy(g_hbm.at[cid, pl.ds(base + j * B, B)],
                             bw[t], sg[t])

        def process(j, t):
            pltpu.make_async_copy(x2_hbm.at[cid].at[idxd_v.at[0]],
                                  bx[t], sg[t]).wait()
            pltpu.make_async_copy(g_hbm.at[cid, pl.ds(0, B)],
                                  bw[t], sg[t]).wait()

            @pl.when(j >= 2)
            def _():
                pltpu.make_async_copy(bs[t], acc_sh.at[idxs_v.at[0]],
                                      ss[t]).wait()

            def row(i, _):
                for l in range(FH // 16):
                    sl = pl.ds(l * 16, 16)
                    bs[t][i, sl] = bx[t][i, sl] * bw[t][i, sl]
                return 0

            lax.fori_loop(0, B, row, 0)

            @pl.when(j + 2 < K2)
            def _():
                gather(j + 2, t)

            pltpu.async_copy(bs[t], acc_sh.at[idxs_v.at[j]], ss[t], add=True)

        gather(0, 0)
        gather(1, 1)

        def pair(p, _):
            process(2 * p, 0)
            process(2 * p + 1, 1)
            return 0

        lax.fori_loop(0, K2 // 2, pair, 0)
        pltpu.make_async_copy(bs[0], acc_sh.at[idxs_v.at[0]], ss[0]).wait()
        pltpu.make_async_copy(bs[1], acc_sh.at[idxs_v.at[0]], ss[1]).wait()
        plsc.subcore_barrier()
        pltpu.sync_copy(acc_sh.at[rows], acc_hbm.at[cid, rows])

    return k(x2h, gate, idx_s2, idx_d2, zeros_n)


def _sc_counts(idx_s2, ones_b, zeros_h):
    @functools.partial(
        pl.kernel,
        out_type=jax.ShapeDtypeStruct((NC, NPAD, FH), jnp.float32),
        mesh=_sc_mesh(),
        compiler_params=pltpu.CompilerParams(use_tc_tiling_on_sc=False),
        scratch_types=[
            pltpu.VMEM((K2, B), jnp.int32),
            pltpu.VMEM((B, FH), jnp.float32),
            pltpu.VMEM_SHARED((NPAD, FH), jnp.float32),
            pltpu.SemaphoreType.DMA,
        ],
    )
    def k(idxs_hbm, ones_hbm, z_hbm, acc_hbm, idxs_v, bones, acc_sh, ss):
        cid = lax.axis_index("c")
        sid = lax.axis_index("s")
        rows = pl.ds(sid * NPT, NPT)
        pltpu.sync_copy(z_hbm.at[rows], acc_sh.at[rows])
        pltpu.sync_copy(idxs_hbm.at[sid], idxs_v)
        pltpu.sync_copy(ones_hbm, bones)
        plsc.subcore_barrier()

        def blk(j, _):
            pltpu.async_copy(bones, acc_sh.at[idxs_v.at[j]], ss, add=True)
            return 0

        lax.fori_loop(0, K2, blk, 0)

        def drain(j, _):
            pltpu.make_async_copy(bones, acc_sh.at[idxs_v.at[0]], ss).wait()
            return 0

        lax.fori_loop(0, K2, drain, 0)
        plsc.subcore_barrier()
        pltpu.sync_copy(acc_sh.at[rows], acc_hbm.at[cid, rows])

    return k(idx_s2, ones_b, zeros_h)


def _tc_init_node(x, W, b):
    def body(x_ref, w_ref, b_ref, o_ref):
        h = jnp.dot(x_ref[...], w_ref[...],
                    preferred_element_type=jnp.float32) + b_ref[...]
        o_ref[...] = h * jax.nn.sigmoid(h)

    return pl.pallas_call(
        body,
        out_shape=jax.ShapeDtypeStruct((N, F), jnp.float32),
    )(x, W, b.reshape(1, F))


def _tc_init_edge(ea, W, b):
    def body(ea_ref, w_ref, b_ref, o_ref, g_ref):
        h = jnp.dot(ea_ref[...], w_ref[...],
                    preferred_element_type=jnp.float32) + b_ref[...]
        o = h * jax.nn.sigmoid(h)
        o_ref[...] = o
        sg = jax.nn.sigmoid(o)
        g_ref[0] = sg[:, :FH]
        g_ref[1] = sg[:, FH:]

    ef = ea.shape[1]
    return pl.pallas_call(
        body,
        grid=(GE,),
        in_specs=[
            pl.BlockSpec((BE, ef), lambda i: (i, 0)),
            pl.BlockSpec((ef, F), lambda i: (0, 0)),
            pl.BlockSpec((1, F), lambda i: (0, 0)),
        ],
        out_specs=(
            pl.BlockSpec((BE, F), lambda i: (i, 0)),
            pl.BlockSpec((NC, BE, FH), lambda i: (0, i, 0)),
        ),
        out_shape=(
            jax.ShapeDtypeStruct((E, F), jnp.float32),
            jax.ShapeDtypeStruct((NC, E, FH), jnp.float32),
        ),
    )(ea, W, b.reshape(1, F))


def _tc_node_mm(x, Wcat, bcat):
    def body(x_ref, w_ref, b_ref, o1, o2, o4, o3):
        h = jnp.dot(x_ref[...], w_ref[...],
                    preferred_element_type=jnp.float32) + b_ref[...]
        o1[...] = h[:, 0 * F:1 * F]
        o2[0] = h[:, F:F + FH]
        o2[1] = h[:, F + FH:2 * F]
        o4[...] = h[:, 2 * F:3 * F]
        o3[...] = h[:, 3 * F:4 * F]

    sh = jax.ShapeDtypeStruct((N, F), jnp.float32)
    sh2 = jax.ShapeDtypeStruct((NC, N, FH), jnp.float32)
    return pl.pallas_call(
        body,
        out_shape=(sh, sh2, sh, sh),
    )(x, Wcat, bcat.reshape(1, 4 * F))


def _tc_edge_pass1(w, s, E0Wi, E0bi):
    def body(w_ref, s_ref, ew_ref, eb_ref, st_ref):
        h = jnp.dot(w_ref[...], ew_ref[...],
                    preferred_element_type=jnp.float32)
        h = h + eb_ref[...] + s_ref[...]

        @pl.when(pl.program_id(0) == 0)
        def _():
            st_ref[...] = jnp.zeros_like(st_ref)

        st_ref[0:1, :] += jnp.sum(h, axis=0, keepdims=True)
        st_ref[1:2, :] += jnp.sum(h * h, axis=0, keepdims=True)

    return pl.pallas_call(
        body,
        grid=(GE,),
        in_specs=[
            pl.BlockSpec((BE, F), lambda i: (i, 0)),
            pl.BlockSpec((BE, F), lambda i: (i, 0)),
            pl.BlockSpec((F, F), lambda i: (0, 0)),
            pl.BlockSpec((1, F), lambda i: (0, 0)),
        ],
        out_specs=pl.BlockSpec((8, F), lambda i: (0, 0)),
        out_shape=jax.ShapeDtypeStruct((8, F), jnp.float32),
    )(w, s, E0Wi, E0bi.reshape(1, F))


def _tc_inv_deg(cnt):
    def body(c_ref, o_ref):
        o_ref[...] = 1.0 / jnp.maximum(c_ref[0, :N], 1.0)

    return pl.pallas_call(
        body,
        out_shape=jax.ShapeDtypeStruct((N, FH), jnp.float32),
    )(cnt)


def _tc_node_update(x0, x1, acc, inv, g, b):
    def body(x0_ref, x1_ref, a_ref, i_ref, g_ref, b_ref, o_ref):
        agg = jnp.concatenate(
            [a_ref[0, :N] * i_ref[...], a_ref[1, :N] * i_ref[...]], axis=1)
        t = x1_ref[...] + agg
        m = jnp.mean(t, axis=0, keepdims=True)
        v = jnp.mean(t * t, axis=0, keepdims=True) - m * m
        y = (t - m) * lax.rsqrt(v + 1e-5) * g_ref[...] + b_ref[...]
        o_ref[...] = x0_ref[...] + y * jax.nn.sigmoid(y)

    return pl.pallas_call(
        body,
        out_shape=jax.ShapeDtypeStruct((N, F), jnp.float32),
    )(x0, x1, acc, inv, g.reshape(1, F), b.reshape(1, F))


def _tc_edge_update(w0, s, E0Wi, E0bi, st, g, b, want_gate):
    def body(w_ref, s_ref, ew_ref, eb_ref, st_ref, g_ref, b_ref,
             o_ref, *rest):
        h = jnp.dot(w_ref[...], ew_ref[...],
                    preferred_element_type=jnp.float32)
        h = h + eb_ref[...] + s_ref[...]
        m = st_ref[0:1, :] * (1.0 / E)
        v = st_ref[1:2, :] * (1.0 / E) - m * m
        y = (h - m) * lax.rsqrt(v + 1e-5) * g_ref[...] + b_ref[...]
        o = w_ref[...] + y * jax.nn.sigmoid(y)
        o_ref[...] = o
        if rest:
            sg = jax.nn.sigmoid(o)
            rest[0][0] = sg[:, :FH]
            rest[0][1] = sg[:, FH:]

    out_specs = [pl.BlockSpec((BE, F), lambda i: (i, 0))]
    out_shape = [jax.ShapeDtypeStruct((E, F), jnp.float32)]
    if want_gate:
        out_specs.append(pl.BlockSpec((NC, BE, FH), lambda i: (0, i, 0)))
        out_shape.append(jax.ShapeDtypeStruct((NC, E, FH), jnp.float32))
    res = pl.pallas_call(
        body,
        grid=(GE,),
        in_specs=[
            pl.BlockSpec((BE, F), lambda i: (i, 0)),
            pl.BlockSpec((BE, F), lambda i: (i, 0)),
            pl.BlockSpec((F, F), lambda i: (0, 0)),
            pl.BlockSpec((1, F), lambda i: (0, 0)),
            pl.BlockSpec((8, F), lambda i: (0, 0)),
            pl.BlockSpec((1, F), lambda i: (0, 0)),
            pl.BlockSpec((1, F), lambda i: (0, 0)),
        ],
        out_specs=tuple(out_specs),
        out_shape=tuple(out_shape),
    )(w0, s, E0Wi, E0bi.reshape(1, F), st, g.reshape(1, F),
      b.reshape(1, F))
    return res if want_gate else (res[0], None)


def kernel(x, edge_index, edge_attr, v_lin0_W, v_lin0_b, V1_W, V1_b, V2_W,
           V2_b, V3_W, V3_b, V4_W, V4_b, vbn_g, vbn_b, e_lin0_W, e_lin0_b,
           E0_W, E0_b, ebn_g, ebn_b):
    src = edge_index[0]
    dst = edge_index[1]
    idx_s3 = src.reshape(NW, K, B)
    idx_d3 = dst.reshape(NW, K, B)
    idx_s2 = src.reshape(NS, K2, B)
    idx_d2 = dst.reshape(NS, K2, B)
    zeros_h = jnp.zeros((NPAD, FH), jnp.float32)
    ones_b = jnp.ones((B, FH), jnp.float32)

    Wcat = jnp.concatenate([V1_W, V2_W, V4_W, V3_W], axis=2)
    bcat = jnp.concatenate([V1_b, V2_b, V4_b, V3_b], axis=1)

    cnt = _sc_counts(idx_s2, ones_b, zeros_h)
    inv = _tc_inv_deg(cnt)

    xc = _tc_init_node(x, v_lin0_W, v_lin0_b)
    wc, gc = _tc_init_edge(edge_attr, e_lin0_W, e_lin0_b)

    for i in range(DEPTH):
        x1, x2h, x4, x3 = _tc_node_mm(xc, Wcat[i], bcat[i])
        s = _sc_gather_sum(x3, x4, idx_s3, idx_d3)
        acc = _sc_gate_scatter(x2h, gc, idx_s2, idx_d2, zeros_h)
        st = _tc_edge_pass1(wc, s, E0_W[i], E0_b[i])
        xc = _tc_node_update(xc, x1, acc, inv, vbn_g[i], vbn_b[i])
        wc, gc = _tc_edge_update(wc, s, E0_W[i], E0_b[i], st,
                                 ebn_g[i], ebn_b[i],
                                 want_gate=(i + 1 < DEPTH))
    return (xc, wc)
